# Initial kernel scaffold; baseline (speedup 1.0000x reference)
import functools

import jax
import jax.numpy as jnp
from jax import lax
from jax.experimental import pallas as pl
from jax.experimental.pallas import tpu as pltpu
from jax.experimental.pallas import tpu_sc as plsc

N = 10000
E = 320000
NC = 2
NS = 16
NW = NC * NS
LANES = 16
EPG = 128
G = 79
EPAD = NW * G * EPG
NPAD = 10016
RPT = N // NS
ZR = 125

_mesh = plsc.VectorSubcoreMesh(
    core_axis_name="c", subcore_axis_name="s", num_cores=NC, num_subcores=NS
)


def _deg_body(dst_hbm, degp_hbm, dstv, ones_v, zb, acc, sem):
    c = lax.axis_index("c")
    s = lax.axis_index("s")
    wid = c * NS + s

    @pl.loop(0, EPG)
    def _fill_ones(i):
        ones_v[i, :] = jnp.ones((LANES,), jnp.float32)

    @pl.loop(0, ZR)
    def _fill_zero(i):
        zb[i, :] = jnp.zeros((LANES,), jnp.float32)

    for r in range(RPT // ZR):
        pltpu.sync_copy(zb, acc.at[pl.ds(s * RPT + r * ZR, ZR)])
    plsc.subcore_barrier()

    pltpu.sync_copy(dst_hbm.at[wid], dstv)

    @pl.loop(0, G)
    def _scatter(g):
        pltpu.sync_copy(ones_v, acc.at[dstv.at[g]], add=True)

    plsc.subcore_barrier()
    pltpu.sync_copy(acc.at[pl.ds(s * RPT, RPT)],
                    degp_hbm.at[c, pl.ds(s * RPT, RPT)])


def _make_deg_kernel():
    return pl.kernel(
        _deg_body,
        out_type=jax.ShapeDtypeStruct((NC, N, LANES), jnp.float32),
        mesh=_mesh,
        scratch_types=[
            pltpu.VMEM((G, EPG), jnp.int32),
            pltpu.VMEM((EPG, LANES), jnp.float32),
            pltpu.VMEM((ZR, LANES), jnp.float32),
            pltpu.VMEM_SHARED((NPAD, LANES), jnp.float32),
            pltpu.SemaphoreType.DMA,
        ],
    )


def _edge_body(w, g_hbm, src_hbm, dst_hbm, part_hbm, srcv, dstv, rows, zb,
               acc, sem):
    c = lax.axis_index("c")
    s = lax.axis_index("s")
    wid = c * NS + s

    pltpu.sync_copy(src_hbm.at[wid], srcv)
    pltpu.sync_copy(dst_hbm.at[wid], dstv)

    @pl.loop(0, ZR)
    def _fill_zero(i):
        for j in range(w // LANES):
            zb[i, pl.ds(j * LANES, LANES)] = jnp.zeros((LANES,), jnp.float32)

    for r in range(RPT // ZR):
        pltpu.sync_copy(zb, acc.at[pl.ds(s * RPT + r * ZR, ZR)])
    plsc.subcore_barrier()

    @pl.loop(0, G)
    def _edges(g):
        pltpu.async_copy(g_hbm.at[srcv.at[g]], rows, sem).wait()
        pltpu.sync_copy(rows, acc.at[dstv.at[g]], add=True)

    plsc.subcore_barrier()
    pltpu.sync_copy(acc.at[pl.ds(s * RPT, RPT)],
                    part_hbm.at[c, pl.ds(s * RPT, RPT)])


def _make_edge_kernel(w):
    return pl.kernel(
        functools.partial(_edge_body, w),
        out_type=jax.ShapeDtypeStruct((NC, N, w), jnp.float32),
        mesh=_mesh,
        scratch_types=[
            pltpu.VMEM((G, EPG), jnp.int32),
            pltpu.VMEM((G, EPG), jnp.int32),
            pltpu.VMEM((EPG, w), jnp.float32),
            pltpu.VMEM((ZR, w), jnp.float32),
            pltpu.VMEM_SHARED((NPAD, w), jnp.float32),
            pltpu.SemaphoreType.DMA,
        ],
    )


_R = 1000


def _l1_body(x_ref, w_ref, d0_ref, d1_ref, g_ref, dis_ref):
    deg = d0_ref[:, :1] + d1_ref[:, :1] + 1.0
    dis = lax.rsqrt(deg)
    dis_ref[...] = dis
    g_ref[...] = dis * jnp.dot(x_ref[...], w_ref[...],
                               preferred_element_type=jnp.float32)


def _mid_body(p0_ref, p1_ref, g_ref, dis_ref, b_ref, w_ref, o_ref):
    t = p0_ref[...] + p1_ref[...] + g_ref[...]
    h = jnp.maximum(dis_ref[...] * t + b_ref[...], 0.0)
    o_ref[...] = dis_ref[...] * jnp.dot(h, w_ref[...],
                                        preferred_element_type=jnp.float32)


def _fin_body(p0_ref, p1_ref, g_ref, dis_ref, b_ref, o_ref):
    t = p0_ref[...] + p1_ref[...] + g_ref[...]
    o_ref[...] = dis_ref[...] * t + b_ref[...]


def _row_spec(w):
    return pl.BlockSpec((_R, w), lambda i: (i, 0))


def _full_spec(a, b):
    return pl.BlockSpec((a, b), lambda i: (0, 0))


def _l1_call(x, w1, d0, d1):
    return pl.pallas_call(
        _l1_body,
        grid=(N // _R,),
        in_specs=[_row_spec(128), _full_spec(128, 128),
                  _row_spec(LANES), _row_spec(LANES)],
        out_specs=[_row_spec(128), _row_spec(1)],
        out_shape=[jax.ShapeDtypeStruct((N, 128), jnp.float32),
                   jax.ShapeDtypeStruct((N, 1), jnp.float32)],
    )(x, w1, d0, d1)


def _mid_call(p0, p1, g, dis, b, w):
    win, wout = w.shape
    return pl.pallas_call(
        _mid_body,
        grid=(N // _R,),
        in_specs=[_row_spec(win), _row_spec(win), _row_spec(win),
                  _row_spec(1), _full_spec(1, win), _full_spec(win, wout)],
        out_specs=_row_spec(wout),
        out_shape=jax.ShapeDtypeStruct((N, wout), jnp.float32),
    )(p0, p1, g, dis, b, w)


def _fin_call(p0, p1, g, dis, b):
    w = g.shape[1]
    return pl.pallas_call(
        _fin_body,
        grid=(N // _R,),
        in_specs=[_row_spec(w), _row_spec(w), _row_spec(w),
                  _row_spec(1), _full_spec(1, w)],
        out_specs=_row_spec(w),
        out_shape=jax.ShapeDtypeStruct((N, w), jnp.float32),
    )(p0, p1, g, dis, b)


@jax.jit
def kernel(x, edge_index, W1, b1, W2, b2, W3, b3, W4, b4):
    src = edge_index[0]
    dst = edge_index[1]
    pad = EPAD - E
    src3 = jnp.concatenate([src, jnp.zeros((pad,), jnp.int32)])
    src3 = src3.reshape(NW, G, EPG)
    dst3 = jnp.concatenate([dst, jnp.full((pad,), N, jnp.int32)])
    dst3 = dst3.reshape(NW, G, EPG)

    degp = _make_deg_kernel()(dst3)

    edge128 = _make_edge_kernel(128)
    edge64 = _make_edge_kernel(64)

    g1, dis = _l1_call(x, W1, degp[0], degp[1])
    p = edge128(g1, src3, dst3)
    g2 = _mid_call(p[0], p[1], g1, dis, b1.reshape(1, -1), W2)
    p = edge128(g2, src3, dst3)
    g3 = _mid_call(p[0], p[1], g2, dis, b2.reshape(1, -1), W3)
    p = edge64(g3, src3, dst3)
    g4 = _mid_call(p[0], p[1], g3, dis, b3.reshape(1, -1), W4)
    p = edge64(g4, src3, dst3)
    return _fin_call(p[0], p[1], g4, dis, b4.reshape(1, -1))


# speedup vs baseline: 13.0995x; 13.0995x over previous
import functools

import jax
import jax.numpy as jnp
from jax import lax
from jax.experimental import pallas as pl
from jax.experimental.pallas import tpu as pltpu
from jax.experimental.pallas import tpu_sc as plsc

N = 10000
NP = 10240
E = 320000
NC = 2
NS = 16
NW = NC * NS
LANES = 16
EPG = 128
GF = 79
GS = 157
RPT = NP // NS
ZR = 128

_mesh = plsc.VectorSubcoreMesh(
    core_axis_name="c", subcore_axis_name="s", num_cores=NC, num_subcores=NS
)


def _zero_acc(zb, acc, s, w):
    @pl.loop(0, ZR)
    def _fill_zero(i):
        for j in range(w // LANES):
            zb[i, pl.ds(j * LANES, LANES)] = jnp.zeros((LANES,), jnp.float32)

    for r in range(RPT // ZR):
        pltpu.sync_copy(zb, acc.at[pl.ds(s * RPT + r * ZR, ZR)])


def _deg_body(dst_hbm, degp_hbm, dstv, ones_v, zb, acc, sem):
    c = lax.axis_index("c")
    s = lax.axis_index("s")
    wid = c * NS + s

    @pl.loop(0, EPG)
    def _fill_ones(i):
        ones_v[i, :] = jnp.ones((LANES,), jnp.float32)

    _zero_acc(zb, acc, s, LANES)
    plsc.subcore_barrier()

    pltpu.sync_copy(dst_hbm.at[wid], dstv)

    @pl.loop(0, GF)
    def _scatter(g):
        pltpu.sync_copy(ones_v, acc.at[dstv.at[g]], add=True)

    plsc.subcore_barrier()
    pltpu.sync_copy(acc.at[pl.ds(s * RPT, RPT)],
                    degp_hbm.at[c, pl.ds(s * RPT, RPT)])


def _make_deg_kernel():
    return pl.kernel(
        _deg_body,
        out_type=jax.ShapeDtypeStruct((NC, NP, LANES), jnp.float32),
        mesh=_mesh,
        compiler_params=pltpu.CompilerParams(use_tc_tiling_on_sc=False),
        scratch_types=[
            pltpu.VMEM((GF, EPG), jnp.int32),
            pltpu.VMEM((EPG, LANES), jnp.float32),
            pltpu.VMEM((ZR, LANES), jnp.float32),
            pltpu.VMEM_SHARED((NP, LANES), jnp.float32),
            pltpu.SemaphoreType.DMA,
        ],
    )


def _edge_full_body(g_hbm, src_hbm, dst_hbm, part_hbm, srcv, dstv, rows, zb,
                    acc, sem):
    c = lax.axis_index("c")
    s = lax.axis_index("s")
    wid = c * NS + s

    pltpu.sync_copy(src_hbm.at[wid], srcv)
    pltpu.sync_copy(dst_hbm.at[wid], dstv)
    _zero_acc(zb, acc, s, 64)
    plsc.subcore_barrier()

    @pl.loop(0, GF)
    def _edges(g):
        pltpu.async_copy(g_hbm.at[srcv.at[g]], rows, sem).wait()
        pltpu.sync_copy(rows, acc.at[dstv.at[g]], add=True)

    plsc.subcore_barrier()
    pltpu.sync_copy(acc.at[pl.ds(s * RPT, RPT)],
                    part_hbm.at[c, pl.ds(s * RPT, RPT)])


def _make_edge_full_kernel():
    return pl.kernel(
        _edge_full_body,
        out_type=jax.ShapeDtypeStruct((NC, NP, 64), jnp.float32),
        mesh=_mesh,
        compiler_params=pltpu.CompilerParams(use_tc_tiling_on_sc=False),
        scratch_types=[
            pltpu.VMEM((GF, EPG), jnp.int32),
            pltpu.VMEM((GF, EPG), jnp.int32),
            pltpu.VMEM((EPG, 64), jnp.float32),
            pltpu.VMEM((ZR, 64), jnp.float32),
            pltpu.VMEM_SHARED((NP, 64), jnp.float32),
            pltpu.SemaphoreType.DMA,
        ],
    )


def _edge_split_body(gcat_hbm, src_hbm, dst_hbm, part_hbm, srcv, dstv, rows,
                     zb, acc, sem):
    c = lax.axis_index("c")
    s = lax.axis_index("s")

    pltpu.sync_copy(src_hbm.at[c, s], srcv)
    pltpu.sync_copy(dst_hbm.at[s], dstv)
    _zero_acc(zb, acc, s, 64)
    plsc.subcore_barrier()

    @pl.loop(0, GS)
    def _edges(g):
        pltpu.async_copy(gcat_hbm.at[srcv.at[g]], rows, sem).wait()
        pltpu.sync_copy(rows, acc.at[dstv.at[g]], add=True)

    plsc.subcore_barrier()
    pltpu.sync_copy(acc.at[pl.ds(s * RPT, RPT)],
                    part_hbm.at[c, pl.ds(s * RPT, RPT)])


def _make_edge_split_kernel():
    return pl.kernel(
        _edge_split_body,
        out_type=jax.ShapeDtypeStruct((NC, NP, 64), jnp.float32),
        mesh=_mesh,
        compiler_params=pltpu.CompilerParams(use_tc_tiling_on_sc=False),
        scratch_types=[
            pltpu.VMEM((GS, EPG), jnp.int32),
            pltpu.VMEM((GS, EPG), jnp.int32),
            pltpu.VMEM((EPG, 64), jnp.float32),
            pltpu.VMEM((ZR, 64), jnp.float32),
            pltpu.VMEM_SHARED((NP, 64), jnp.float32),
            pltpu.SemaphoreType.DMA,
        ],
    )


_R = 1024


def _l1_body(x_ref, w_ref, d0_ref, d1_ref, g_ref, dis_ref):
    deg = d0_ref[:, :1] + d1_ref[:, :1] + 1.0
    dis = lax.rsqrt(deg)
    dis_ref[...] = dis
    g = dis * jnp.dot(x_ref[...], w_ref[...],
                      preferred_element_type=jnp.float32)
    g_ref[0] = g[:, :64]
    g_ref[1] = g[:, 64:]


def _mid_split_body(split_out, p_ref, g_ref, dis_ref, b_ref, w_ref, o_ref):
    dis = dis_ref[...]
    h0 = jnp.maximum(dis * (p_ref[0] + g_ref[0]) + b_ref[:, :64], 0.0)
    h1 = jnp.maximum(dis * (p_ref[1] + g_ref[1]) + b_ref[:, 64:], 0.0)
    g = dis * (jnp.dot(h0, w_ref[:64], preferred_element_type=jnp.float32)
               + jnp.dot(h1, w_ref[64:], preferred_element_type=jnp.float32))
    if split_out:
        o_ref[0] = g[:, :64]
        o_ref[1] = g[:, 64:]
    else:
        o_ref[...] = g


def _mid_full_body(p_ref, g_ref, dis_ref, b_ref, w_ref, o_ref):
    dis = dis_ref[...]
    t = p_ref[0] + p_ref[1] + g_ref[...]
    h = jnp.maximum(dis * t + b_ref[...], 0.0)
    o_ref[...] = dis * jnp.dot(h, w_ref[...],
                               preferred_element_type=jnp.float32)


def _fin_body(p_ref, g_ref, dis_ref, b_ref, o_ref):
    t = p_ref[0] + p_ref[1] + g_ref[...]
    o_ref[...] = dis_ref[...] * t + b_ref[...]


def _row_spec(w):
    return pl.BlockSpec((_R, w), lambda i: (i, 0))


def _half_spec():
    return pl.BlockSpec((NC, _R, 64), lambda i: (0, i, 0))


def _full_spec(a, b):
    return pl.BlockSpec((a, b), lambda i: (0, 0))


def _l1_call(x, w1, d0, d1):
    return pl.pallas_call(
        _l1_body,
        grid=(NP // _R,),
        in_specs=[_row_spec(128), _full_spec(128, 128),
                  _row_spec(LANES), _row_spec(LANES)],
        out_specs=[_half_spec(), _row_spec(1)],
        out_shape=[jax.ShapeDtypeStruct((NC, NP, 64), jnp.float32),
                   jax.ShapeDtypeStruct((NP, 1), jnp.float32)],
    )(x, w1, d0, d1)


def _mid_split_call(p, g, dis, b, w, split_out):
    wout = w.shape[1]
    out_spec = _half_spec() if split_out else _row_spec(wout)
    out_shape = (jax.ShapeDtypeStruct((NC, NP, 64), jnp.float32) if split_out
                 else jax.ShapeDtypeStruct((NP, wout), jnp.float32))
    return pl.pallas_call(
        functools.partial(_mid_split_body, split_out),
        grid=(NP // _R,),
        in_specs=[_half_spec(), _half_spec(), _row_spec(1),
                  _full_spec(1, 128), _full_spec(128, wout)],
        out_specs=out_spec,
        out_shape=out_shape,
    )(p, g, dis, b, w)


def _mid_full_call(p, g, dis, b, w):
    wout = w.shape[1]
    return pl.pallas_call(
        _mid_full_body,
        grid=(NP // _R,),
        in_specs=[_half_spec(), _row_spec(64), _row_spec(1),
                  _full_spec(1, 64), _full_spec(64, wout)],
        out_specs=_row_spec(wout),
        out_shape=jax.ShapeDtypeStruct((NP, wout), jnp.float32),
    )(p, g, dis, b, w)


def _fin_call(p, g, dis, b):
    return pl.pallas_call(
        _fin_body,
        grid=(NP // _R,),
        in_specs=[_half_spec(), _row_spec(64), _row_spec(1),
                  _full_spec(1, 64)],
        out_specs=_row_spec(64),
        out_shape=jax.ShapeDtypeStruct((NP, 64), jnp.float32),
    )(p, g, dis, b)


@jax.jit
def kernel(x, edge_index, W1, b1, W2, b2, W3, b3, W4, b4):
    src = edge_index[0]
    dst = edge_index[1]
    xp = jnp.concatenate([x, jnp.zeros((NP - N, x.shape[1]), x.dtype)])

    padf = NW * GF * EPG - E
    src_f = jnp.concatenate([src, jnp.zeros((padf,), jnp.int32)])
    src_f = src_f.reshape(NW, GF, EPG)
    dst_f = jnp.concatenate([dst, jnp.full((padf,), N, jnp.int32)])
    dst_f = dst_f.reshape(NW, GF, EPG)

    pads = NS * GS * EPG - E
    src_s0 = jnp.concatenate([src, jnp.zeros((pads,), jnp.int32)])
    src_s0 = src_s0.reshape(NS, GS, EPG)
    src_s = jnp.stack([src_s0, src_s0 + NP])
    dst_s = jnp.concatenate([dst, jnp.full((pads,), N, jnp.int32)])
    dst_s = dst_s.reshape(NS, GS, EPG)

    degp = _make_deg_kernel()(dst_f)
    edge_split = _make_edge_split_kernel()
    edge_full = _make_edge_full_kernel()

    g1, dis = _l1_call(xp, W1, degp[0], degp[1])
    p = edge_split(g1.reshape(NC * NP, 64), src_s, dst_s)
    g2 = _mid_split_call(p, g1, dis, b1.reshape(1, -1), W2, True)
    p = edge_split(g2.reshape(NC * NP, 64), src_s, dst_s)
    g3 = _mid_split_call(p, g2, dis, b2.reshape(1, -1), W3, False)
    p = edge_full(g3, src_f, dst_f)
    g4 = _mid_full_call(p, g3, dis, b3.reshape(1, -1), W4)
    p = edge_full(g4, src_f, dst_f)
    return _fin_call(p, g4, dis, b4.reshape(1, -1))[:N]


# speedup vs baseline: 27.3946x; 2.0913x over previous
import functools

import jax
import jax.numpy as jnp
from jax import lax
from jax.experimental import pallas as pl
from jax.experimental.pallas import tpu as pltpu
from jax.experimental.pallas import tpu_sc as plsc

N = 10000
NP = 10240
E = 320000
NC = 2
NS = 16
NW = NC * NS
LANES = 16
EPG = 128
GF = 80
GS = 160
RPT = NP // NS
ZR = 64

_mesh = plsc.VectorSubcoreMesh(
    core_axis_name="c", subcore_axis_name="s", num_cores=NC, num_subcores=NS
)


def _zero_acc(zb, acc, s, w):
    @pl.loop(0, ZR)
    def _fill_zero(i):
        for j in range(w // LANES):
            zb[i, pl.ds(j * LANES, LANES)] = jnp.zeros((LANES,), jnp.float32)

    for r in range(RPT // ZR):
        pltpu.sync_copy(zb, acc.at[pl.ds(s * RPT + r * ZR, ZR)])


def _edge_pipeline(table, srcv, dstv, acc, rows4, gsem, ssem, G):
    K = G // 4
    pltpu.async_copy(table.at[srcv.at[0]], rows4[0], gsem)
    pltpu.async_copy(table.at[srcv.at[1]], rows4[1], gsem)

    @pl.loop(0, K)
    def _groups(k):
        for j in range(4):
            gj = k * 4 + j
            r = rows4[j]
            pltpu.make_async_copy(table.at[srcv.at[gj]], r, gsem).wait()
            pltpu.async_copy(r, acc.at[dstv.at[gj]], ssem, add=True)
            b = rows4[(j + 2) % 4]
            if j < 2:
                @pl.when(k > 0)
                def _drain(b=b, gj=gj):
                    pltpu.make_async_copy(
                        b, acc.at[dstv.at[gj - 2]], ssem).wait()
                pltpu.async_copy(table.at[srcv.at[gj + 2]], b, gsem)
            else:
                @pl.when(k < K - 1)
                def _refill(b=b, gj=gj):
                    pltpu.make_async_copy(
                        b, acc.at[dstv.at[gj - 2]], ssem).wait()
                    pltpu.async_copy(table.at[srcv.at[gj + 2]], b, gsem)

    for g in range(G - 4, G):
        pltpu.make_async_copy(rows4[g % 4], acc.at[dstv.at[g]], ssem).wait()


def _deg_body(dst_hbm, degp_hbm, dstv, ones_v, zb, acc, sem):
    c = lax.axis_index("c")
    s = lax.axis_index("s")
    wid = c * NS + s

    @pl.loop(0, EPG)
    def _fill_ones(i):
        ones_v[i, :] = jnp.ones((LANES,), jnp.float32)

    _zero_acc(zb, acc, s, LANES)
    plsc.subcore_barrier()

    pltpu.sync_copy(dst_hbm.at[wid], dstv)

    @pl.loop(0, GF)
    def _scatter(g):
        pltpu.sync_copy(ones_v, acc.at[dstv.at[g]], add=True)

    plsc.subcore_barrier()
    pltpu.sync_copy(acc.at[pl.ds(s * RPT, RPT)],
                    degp_hbm.at[c, pl.ds(s * RPT, RPT)])


def _make_deg_kernel():
    return pl.kernel(
        _deg_body,
        out_type=jax.ShapeDtypeStruct((NC, NP, LANES), jnp.float32),
        mesh=_mesh,
        compiler_params=pltpu.CompilerParams(use_tc_tiling_on_sc=False),
        scratch_types=[
            pltpu.VMEM((GF, EPG), jnp.int32),
            pltpu.VMEM((EPG, LANES), jnp.float32),
            pltpu.VMEM((ZR, LANES), jnp.float32),
            pltpu.VMEM_SHARED((NP, LANES), jnp.float32),
            pltpu.SemaphoreType.DMA,
        ],
    )


def _edge_full_body(g_hbm, src_hbm, dst_hbm, part_hbm, srcv, dstv, rows, zb,
                    acc, gsem, ssem):
    c = lax.axis_index("c")
    s = lax.axis_index("s")
    wid = c * NS + s

    pltpu.sync_copy(src_hbm.at[wid], srcv)
    pltpu.sync_copy(dst_hbm.at[wid], dstv)
    _zero_acc(zb, acc, s, 64)
    plsc.subcore_barrier()

    rows4 = [rows.at[j] for j in range(4)]
    _edge_pipeline(g_hbm, srcv, dstv, acc, rows4, gsem, ssem, GF)

    plsc.subcore_barrier()
    pltpu.sync_copy(acc.at[pl.ds(s * RPT, RPT)],
                    part_hbm.at[c, pl.ds(s * RPT, RPT)])


def _make_edge_full_kernel():
    return pl.kernel(
        _edge_full_body,
        out_type=jax.ShapeDtypeStruct((NC, NP, 64), jnp.float32),
        mesh=_mesh,
        compiler_params=pltpu.CompilerParams(use_tc_tiling_on_sc=False),
        scratch_types=[
            pltpu.VMEM((GF, EPG), jnp.int32),
            pltpu.VMEM((GF, EPG), jnp.int32),
            pltpu.VMEM((4, EPG, 64), jnp.float32),
            pltpu.VMEM((ZR, 64), jnp.float32),
            pltpu.VMEM_SHARED((NP, 64), jnp.float32),
            pltpu.SemaphoreType.DMA,
            pltpu.SemaphoreType.DMA,
        ],
    )


def _edge_split_body(gcat_hbm, src_hbm, dst_hbm, part_hbm, srcv, dstv, rows,
                     zb, acc, gsem, ssem):
    c = lax.axis_index("c")
    s = lax.axis_index("s")

    pltpu.sync_copy(src_hbm.at[c, s], srcv)
    pltpu.sync_copy(dst_hbm.at[s], dstv)
    _zero_acc(zb, acc, s, 64)
    plsc.subcore_barrier()

    rows4 = [rows.at[j] for j in range(4)]
    _edge_pipeline(gcat_hbm, srcv, dstv, acc, rows4, gsem, ssem, GS)

    plsc.subcore_barrier()
    pltpu.sync_copy(acc.at[pl.ds(s * RPT, RPT)],
                    part_hbm.at[c, pl.ds(s * RPT, RPT)])


def _make_edge_split_kernel():
    return pl.kernel(
        _edge_split_body,
        out_type=jax.ShapeDtypeStruct((NC, NP, 64), jnp.float32),
        mesh=_mesh,
        compiler_params=pltpu.CompilerParams(use_tc_tiling_on_sc=False),
        scratch_types=[
            pltpu.VMEM((GS, EPG), jnp.int32),
            pltpu.VMEM((GS, EPG), jnp.int32),
            pltpu.VMEM((4, EPG, 64), jnp.float32),
            pltpu.VMEM((ZR, 64), jnp.float32),
            pltpu.VMEM_SHARED((NP, 64), jnp.float32),
            pltpu.SemaphoreType.DMA,
            pltpu.SemaphoreType.DMA,
        ],
    )


_R = 1024


def _l1_body(x_ref, w_ref, d0_ref, d1_ref, g_ref, dis_ref):
    deg = d0_ref[:, :1] + d1_ref[:, :1] + 1.0
    dis = lax.rsqrt(deg)
    dis_ref[...] = dis
    g = dis * jnp.dot(x_ref[...], w_ref[...],
                      preferred_element_type=jnp.float32)
    g_ref[0] = g[:, :64]
    g_ref[1] = g[:, 64:]


def _mid_split_body(split_out, p_ref, g_ref, dis_ref, b_ref, w_ref, o_ref):
    dis = dis_ref[...]
    h0 = jnp.maximum(dis * (p_ref[0] + g_ref[0]) + b_ref[:, :64], 0.0)
    h1 = jnp.maximum(dis * (p_ref[1] + g_ref[1]) + b_ref[:, 64:], 0.0)
    g = dis * (jnp.dot(h0, w_ref[:64], preferred_element_type=jnp.float32)
               + jnp.dot(h1, w_ref[64:], preferred_element_type=jnp.float32))
    if split_out:
        o_ref[0] = g[:, :64]
        o_ref[1] = g[:, 64:]
    else:
        o_ref[...] = g


def _mid_full_body(p_ref, g_ref, dis_ref, b_ref, w_ref, o_ref):
    dis = dis_ref[...]
    t = p_ref[0] + p_ref[1] + g_ref[...]
    h = jnp.maximum(dis * t + b_ref[...], 0.0)
    o_ref[...] = dis * jnp.dot(h, w_ref[...],
                               preferred_element_type=jnp.float32)


def _fin_body(p_ref, g_ref, dis_ref, b_ref, o_ref):
    t = p_ref[0] + p_ref[1] + g_ref[...]
    o_ref[...] = dis_ref[...] * t + b_ref[...]


def _row_spec(w):
    return pl.BlockSpec((_R, w), lambda i: (i, 0))


def _half_spec():
    return pl.BlockSpec((NC, _R, 64), lambda i: (0, i, 0))


def _full_spec(a, b):
    return pl.BlockSpec((a, b), lambda i: (0, 0))


def _l1_call(x, w1, d0, d1):
    return pl.pallas_call(
        _l1_body,
        grid=(NP // _R,),
        in_specs=[_row_spec(128), _full_spec(128, 128),
                  _row_spec(LANES), _row_spec(LANES)],
        out_specs=[_half_spec(), _row_spec(1)],
        out_shape=[jax.ShapeDtypeStruct((NC, NP, 64), jnp.float32),
                   jax.ShapeDtypeStruct((NP, 1), jnp.float32)],
    )(x, w1, d0, d1)


def _mid_split_call(p, g, dis, b, w, split_out):
    wout = w.shape[1]
    out_spec = _half_spec() if split_out else _row_spec(wout)
    out_shape = (jax.ShapeDtypeStruct((NC, NP, 64), jnp.float32) if split_out
                 else jax.ShapeDtypeStruct((NP, wout), jnp.float32))
    return pl.pallas_call(
        functools.partial(_mid_split_body, split_out),
        grid=(NP // _R,),
        in_specs=[_half_spec(), _half_spec(), _row_spec(1),
                  _full_spec(1, 128), _full_spec(128, wout)],
        out_specs=out_spec,
        out_shape=out_shape,
    )(p, g, dis, b, w)


def _mid_full_call(p, g, dis, b, w):
    wout = w.shape[1]
    return pl.pallas_call(
        _mid_full_body,
        grid=(NP // _R,),
        in_specs=[_half_spec(), _row_spec(64), _row_spec(1),
                  _full_spec(1, 64), _full_spec(64, wout)],
        out_specs=_row_spec(wout),
        out_shape=jax.ShapeDtypeStruct((NP, wout), jnp.float32),
    )(p, g, dis, b, w)


def _fin_call(p, g, dis, b):
    return pl.pallas_call(
        _fin_body,
        grid=(NP // _R,),
        in_specs=[_half_spec(), _row_spec(64), _row_spec(1),
                  _full_spec(1, 64)],
        out_specs=_row_spec(64),
        out_shape=jax.ShapeDtypeStruct((NP, 64), jnp.float32),
    )(p, g, dis, b)


@jax.jit
def kernel(x, edge_index, W1, b1, W2, b2, W3, b3, W4, b4):
    src = edge_index[0]
    dst = edge_index[1]
    xp = jnp.concatenate([x, jnp.zeros((NP - N, x.shape[1]), x.dtype)])

    padf = NW * GF * EPG - E
    pad_src = (jnp.arange(padf, dtype=jnp.int32) * 37) % N
    pad_dst = N + jnp.arange(padf, dtype=jnp.int32) % (NP - N)
    src_f = jnp.concatenate([src, pad_src]).reshape(NW, GF, EPG)
    dst_f = jnp.concatenate([dst, pad_dst]).reshape(NW, GF, EPG)

    src_s0 = jnp.concatenate([src, pad_src]).reshape(NS, GS, EPG)
    src_s = jnp.stack([src_s0, src_s0 + NP])
    dst_s = jnp.concatenate([dst, pad_dst]).reshape(NS, GS, EPG)

    degp = _make_deg_kernel()(dst_f)
    edge_split = _make_edge_split_kernel()
    edge_full = _make_edge_full_kernel()

    g1, dis = _l1_call(xp, W1, degp[0], degp[1])
    p = edge_split(g1.reshape(NC * NP, 64), src_s, dst_s)
    g2 = _mid_split_call(p, g1, dis, b1.reshape(1, -1), W2, True)
    p = edge_split(g2.reshape(NC * NP, 64), src_s, dst_s)
    g3 = _mid_split_call(p, g2, dis, b2.reshape(1, -1), W3, False)
    p = edge_full(g3, src_f, dst_f)
    g4 = _mid_full_call(p, g3, dis, b3.reshape(1, -1), W4)
    p = edge_full(g4, src_f, dst_f)
    return _fin_call(p, g4, dis, b4.reshape(1, -1))[:N]


# speedup vs baseline: 27.8986x; 1.0184x over previous
import functools

import jax
import jax.numpy as jnp
from jax import lax
from jax.experimental import pallas as pl
from jax.experimental.pallas import tpu as pltpu
from jax.experimental.pallas import tpu_sc as plsc

N = 10000
NP = 10240
E = 320000
NC = 2
NS = 16
NW = NC * NS
LANES = 16
EPG = 128
GF = 80
GS = 160
RPT = NP // NS
ZR = 64

_mesh = plsc.VectorSubcoreMesh(
    core_axis_name="c", subcore_axis_name="s", num_cores=NC, num_subcores=NS
)


def _zero_acc(zb, acc, s, w):
    @pl.loop(0, ZR)
    def _fill_zero(i):
        for j in range(w // LANES):
            zb[i, pl.ds(j * LANES, LANES)] = jnp.zeros((LANES,), jnp.float32)

    for r in range(RPT // ZR):
        pltpu.sync_copy(zb, acc.at[pl.ds(s * RPT + r * ZR, ZR)])


def _edge_pipeline(table, srcv, dstv, acc, rows4, gsem, ssem, G):
    K = G // 4
    pltpu.async_copy(table.at[srcv.at[0]], rows4[0], gsem)
    pltpu.async_copy(table.at[srcv.at[1]], rows4[1], gsem)

    @pl.loop(0, K)
    def _groups(k):
        for j in range(4):
            gj = k * 4 + j
            r = rows4[j]
            pltpu.make_async_copy(table.at[srcv.at[gj]], r, gsem).wait()
            pltpu.async_copy(r, acc.at[dstv.at[gj]], ssem, add=True)
            b = rows4[(j + 2) % 4]
            if j < 2:
                @pl.when(k > 0)
                def _drain(b=b, gj=gj):
                    pltpu.make_async_copy(
                        b, acc.at[dstv.at[gj - 2]], ssem).wait()
                pltpu.async_copy(table.at[srcv.at[gj + 2]], b, gsem)
            else:
                @pl.when(k < K - 1)
                def _refill(b=b, gj=gj):
                    pltpu.make_async_copy(
                        b, acc.at[dstv.at[gj - 2]], ssem).wait()
                    pltpu.async_copy(table.at[srcv.at[gj + 2]], b, gsem)

    for g in range(G - 4, G):
        pltpu.make_async_copy(rows4[g % 4], acc.at[dstv.at[g]], ssem).wait()


def _deg_body(dst_hbm, degp_hbm, dstv, ones_v, zb, acc, sem):
    c = lax.axis_index("c")
    s = lax.axis_index("s")
    wid = c * NS + s

    @pl.loop(0, EPG)
    def _fill_ones(i):
        ones_v[i, :] = jnp.ones((LANES,), jnp.float32)

    _zero_acc(zb, acc, s, LANES)
    plsc.subcore_barrier()

    pltpu.sync_copy(dst_hbm.at[wid], dstv)

    @pl.loop(0, GF)
    def _scatter(g):
        pltpu.sync_copy(ones_v, acc.at[dstv.at[g]], add=True)

    plsc.subcore_barrier()
    pltpu.sync_copy(acc.at[pl.ds(s * RPT, RPT)],
                    degp_hbm.at[c, pl.ds(s * RPT, RPT)])


def _make_deg_kernel():
    return pl.kernel(
        _deg_body,
        out_type=jax.ShapeDtypeStruct((NC, NP, LANES), jnp.float32),
        mesh=_mesh,
        compiler_params=pltpu.CompilerParams(use_tc_tiling_on_sc=False),
        scratch_types=[
            pltpu.VMEM((GF, EPG), jnp.int32),
            pltpu.VMEM((EPG, LANES), jnp.float32),
            pltpu.VMEM((ZR, LANES), jnp.float32),
            pltpu.VMEM_SHARED((NP, LANES), jnp.float32),
            pltpu.SemaphoreType.DMA,
        ],
    )


def _edge_body(hw, gv_hbm, src_hbm, dst_hbm, out_hbm, srcv, dstv, rows, zb,
               acc, gsem, ssem):
    c = lax.axis_index("c")
    s = lax.axis_index("s")

    pltpu.sync_copy(src_hbm.at[c, s], srcv)
    pltpu.sync_copy(dst_hbm.at[s], dstv)
    _zero_acc(zb, acc, s, hw)
    plsc.subcore_barrier()

    rows4 = [rows.at[j] for j in range(4)]
    _edge_pipeline(gv_hbm, srcv, dstv, acc, rows4, gsem, ssem, GS)

    plsc.subcore_barrier()
    pltpu.sync_copy(acc.at[pl.ds(s * RPT, RPT)],
                    out_hbm.at[pl.ds(s * RPT, RPT), pl.ds(c * hw, hw)])


def _make_edge_kernel(w):
    hw = w // 2
    return pl.kernel(
        functools.partial(_edge_body, hw),
        out_type=jax.ShapeDtypeStruct((NP, w), jnp.float32),
        mesh=_mesh,
        compiler_params=pltpu.CompilerParams(use_tc_tiling_on_sc=False),
        scratch_types=[
            pltpu.VMEM((GS, EPG), jnp.int32),
            pltpu.VMEM((GS, EPG), jnp.int32),
            pltpu.VMEM((4, EPG, hw), jnp.float32),
            pltpu.VMEM((ZR, hw), jnp.float32),
            pltpu.VMEM_SHARED((NP, hw), jnp.float32),
            pltpu.SemaphoreType.DMA,
            pltpu.SemaphoreType.DMA,
        ],
    )


_R = 1024


def _l1_body(x_ref, w_ref, d0_ref, d1_ref, g_ref, dis_ref):
    deg = d0_ref[:, :1] + d1_ref[:, :1] + 1.0
    dis = lax.rsqrt(deg)
    dis_ref[...] = dis
    g_ref[...] = dis * jnp.dot(x_ref[...], w_ref[...],
                               preferred_element_type=jnp.float32)


def _mid_body(p_ref, g_ref, dis_ref, b_ref, w_ref, o_ref):
    dis = dis_ref[...]
    h = jnp.maximum(dis * (p_ref[...] + g_ref[...]) + b_ref[...], 0.0)
    o_ref[...] = dis * jnp.dot(h, w_ref[...],
                               preferred_element_type=jnp.float32)


def _fin_body(p_ref, g_ref, dis_ref, b_ref, o_ref):
    o_ref[...] = dis_ref[...] * (p_ref[...] + g_ref[...]) + b_ref[...]


def _row_spec(w):
    return pl.BlockSpec((_R, w), lambda i: (i, 0))


def _full_spec(a, b):
    return pl.BlockSpec((a, b), lambda i: (0, 0))


def _l1_call(x, w1, d0, d1):
    return pl.pallas_call(
        _l1_body,
        grid=(NP // _R,),
        in_specs=[_row_spec(128), _full_spec(128, 128),
                  _row_spec(LANES), _row_spec(LANES)],
        out_specs=[_row_spec(128), _row_spec(1)],
        out_shape=[jax.ShapeDtypeStruct((NP, 128), jnp.float32),
                   jax.ShapeDtypeStruct((NP, 1), jnp.float32)],
    )(x, w1, d0, d1)


def _mid_call(p, g, dis, b, w):
    win, wout = w.shape
    return pl.pallas_call(
        _mid_body,
        grid=(NP // _R,),
        in_specs=[_row_spec(win), _row_spec(win), _row_spec(1),
                  _full_spec(1, win), _full_spec(win, wout)],
        out_specs=_row_spec(wout),
        out_shape=jax.ShapeDtypeStruct((NP, wout), jnp.float32),
    )(p, g, dis, b, w)


_RF = 1000


def _fin_call(p, g, dis, b):
    return pl.pallas_call(
        _fin_body,
        grid=(N // _RF,),
        in_specs=[pl.BlockSpec((_RF, 64), lambda i: (i, 0)),
                  pl.BlockSpec((_RF, 64), lambda i: (i, 0)),
                  pl.BlockSpec((_RF, 1), lambda i: (i, 0)),
                  _full_spec(1, 64)],
        out_specs=pl.BlockSpec((_RF, 64), lambda i: (i, 0)),
        out_shape=jax.ShapeDtypeStruct((N, 64), jnp.float32),
    )(p, g, dis, b)


@jax.jit
def kernel(x, edge_index, W1, b1, W2, b2, W3, b3, W4, b4):
    src = edge_index[0]
    dst = edge_index[1]
    xp = jnp.concatenate([x, jnp.zeros((NP - N, x.shape[1]), x.dtype)])

    pad = NS * GS * EPG - E
    pad_src = (jnp.arange(pad, dtype=jnp.int32) * 37) % N
    pad_dst = N + jnp.arange(pad, dtype=jnp.int32) % (NP - N)
    srcp = 2 * jnp.concatenate([src, pad_src])
    src_s = jnp.stack([srcp, srcp + 1]).reshape(NC, NS, GS, EPG)
    dst_sp = jnp.concatenate([dst, pad_dst])
    dst_s = dst_sp.reshape(NS, GS, EPG)
    dst_f = dst_sp.reshape(NW, GF, EPG)

    degp = _make_deg_kernel()(dst_f)
    edge128 = _make_edge_kernel(128)
    edge64 = _make_edge_kernel(64)

    g1, dis = _l1_call(xp, W1, degp[0], degp[1])
    p = edge128(g1.reshape(2 * NP, 64), src_s, dst_s)
    g2 = _mid_call(p, g1, dis, b1.reshape(1, -1), W2)
    p = edge128(g2.reshape(2 * NP, 64), src_s, dst_s)
    g3 = _mid_call(p, g2, dis, b2.reshape(1, -1), W3)
    p = edge64(g3.reshape(2 * NP, 32), src_s, dst_s)
    g4 = _mid_call(p, g3, dis, b3.reshape(1, -1), W4)
    p = edge64(g4.reshape(2 * NP, 32), src_s, dst_s)
    return _fin_call(p, g4, dis, b4.reshape(1, -1))


# speedup vs baseline: 31.1108x; 1.1151x over previous
import functools

import jax
import jax.numpy as jnp
from jax import lax
from jax.experimental import pallas as pl
from jax.experimental.pallas import tpu as pltpu
from jax.experimental.pallas import tpu_sc as plsc

N = 10000
NP = 10240
NM = NP // 2
E = 320000
NC = 2
NS = 16
LANES = 16
EPG = 128
GS = 160
GH = GS // 2
RPT = NP // NS
ZR = 64

_mesh = plsc.VectorSubcoreMesh(
    core_axis_name="c", subcore_axis_name="s", num_cores=NC, num_subcores=NS
)


def _zero_acc(zb, acc, s, w):
    @pl.loop(0, ZR)
    def _fill_zero(i):
        for j in range(w // LANES):
            zb[i, pl.ds(j * LANES, LANES)] = jnp.zeros((LANES,), jnp.float32)

    for r in range(RPT // ZR):
        pltpu.sync_copy(zb, acc.at[pl.ds(s * RPT + r * ZR, ZR)])


def _edge_pipeline(table, srcv, dstv, acc, rows4, gsem, ssem, G):
    K = G // 4
    pltpu.async_copy(table.at[srcv.at[0]], rows4[0], gsem)
    pltpu.async_copy(table.at[srcv.at[1]], rows4[1], gsem)

    @pl.loop(0, K)
    def _groups(k):
        for j in range(4):
            gj = k * 4 + j
            r = rows4[j]
            pltpu.make_async_copy(table.at[srcv.at[gj]], r, gsem).wait()
            pltpu.async_copy(r, acc.at[dstv.at[gj]], ssem, add=True)
            b = rows4[(j + 2) % 4]
            if j < 2:
                @pl.when(k > 0)
                def _drain(b=b, gj=gj):
                    pltpu.make_async_copy(
                        b, acc.at[dstv.at[gj - 2]], ssem).wait()
                pltpu.async_copy(table.at[srcv.at[gj + 2]], b, gsem)
            else:
                @pl.when(k < K - 1)
                def _refill(b=b, gj=gj):
                    pltpu.make_async_copy(
                        b, acc.at[dstv.at[gj - 2]], ssem).wait()
                    pltpu.async_copy(table.at[srcv.at[gj + 2]], b, gsem)

    for g in range(G - 4, G):
        pltpu.make_async_copy(rows4[g % 4], acc.at[dstv.at[g]], ssem).wait()


def _deg_body(dst_hbm, degp_hbm, dstv, ones_v, zb, acc, sem):
    c = lax.axis_index("c")
    s = lax.axis_index("s")

    @pl.loop(0, EPG)
    def _fill_ones(i):
        ones_v[i, :] = jnp.ones((LANES,), jnp.float32)

    _zero_acc(zb, acc, s, LANES)
    plsc.subcore_barrier()

    pltpu.sync_copy(dst_hbm.at[s, pl.ds(c * GH, GH)], dstv)

    @pl.loop(0, GH)
    def _scatter(g):
        pltpu.sync_copy(ones_v, acc.at[dstv.at[g]], add=True)

    plsc.subcore_barrier()
    pltpu.sync_copy(acc.at[pl.ds(s * RPT, RPT)],
                    degp_hbm.at[c, pl.ds(s * RPT, RPT)])


def _make_deg_kernel():
    return pl.kernel(
        _deg_body,
        out_type=jax.ShapeDtypeStruct((NC, NP, LANES), jnp.float32),
        mesh=_mesh,
        compiler_params=pltpu.CompilerParams(use_tc_tiling_on_sc=False),
        scratch_types=[
            pltpu.VMEM((GH, EPG), jnp.int32),
            pltpu.VMEM((EPG, LANES), jnp.float32),
            pltpu.VMEM((ZR, LANES), jnp.float32),
            pltpu.VMEM_SHARED((NP, LANES), jnp.float32),
            pltpu.SemaphoreType.DMA,
        ],
    )


def _edge128_body(gv_hbm, src_hbm, dst_hbm, out_hbm, srcv, dstv, rows, zb,
                  acc, gsem, ssem):
    c = lax.axis_index("c")
    s = lax.axis_index("s")

    pltpu.sync_copy(src_hbm.at[s], srcv)
    pltpu.sync_copy(dst_hbm.at[s], dstv)

    @pl.loop(0, GS)
    def _double(i):
        for j in range(EPG // LANES):
            sl = pl.ds(j * LANES, LANES)
            srcv[i, sl] = srcv[i, sl] * 2 + c

    _zero_acc(zb, acc, s, 64)
    plsc.subcore_barrier()

    rows4 = [rows.at[j] for j in range(4)]
    _edge_pipeline(gv_hbm, srcv, dstv, acc, rows4, gsem, ssem, GS)

    plsc.subcore_barrier()
    pltpu.sync_copy(acc.at[pl.ds(s * RPT, RPT)],
                    out_hbm.at[pl.ds(s * RPT, RPT), pl.ds(c * 64, 64)])


def _make_edge128_kernel():
    return pl.kernel(
        _edge128_body,
        out_type=jax.ShapeDtypeStruct((NP, 128), jnp.float32),
        mesh=_mesh,
        compiler_params=pltpu.CompilerParams(use_tc_tiling_on_sc=False),
        scratch_types=[
            pltpu.VMEM((GS, EPG), jnp.int32),
            pltpu.VMEM((GS, EPG), jnp.int32),
            pltpu.VMEM((4, EPG, 64), jnp.float32),
            pltpu.VMEM((ZR, 64), jnp.float32),
            pltpu.VMEM_SHARED((NP, 64), jnp.float32),
            pltpu.SemaphoreType.DMA,
            pltpu.SemaphoreType.DMA,
        ],
    )


def _edge64_body(g_hbm, src_hbm, dst_hbm, part_hbm, srcv, dstv, rows, zb,
                 acc, gsem, ssem):
    c = lax.axis_index("c")
    s = lax.axis_index("s")

    pltpu.sync_copy(src_hbm.at[s, pl.ds(c * GH, GH)], srcv)
    pltpu.sync_copy(dst_hbm.at[s, pl.ds(c * GH, GH)], dstv)
    _zero_acc(zb, acc, s, 64)
    plsc.subcore_barrier()

    rows4 = [rows.at[j] for j in range(4)]
    _edge_pipeline(g_hbm, srcv, dstv, acc, rows4, gsem, ssem, GH)

    plsc.subcore_barrier()
    pltpu.sync_copy(acc.at[pl.ds(s * RPT, RPT)],
                    part_hbm.at[c, pl.ds(s * RPT, RPT)])


def _make_edge64_kernel():
    return pl.kernel(
        _edge64_body,
        out_type=jax.ShapeDtypeStruct((NC, NP, 64), jnp.float32),
        mesh=_mesh,
        compiler_params=pltpu.CompilerParams(use_tc_tiling_on_sc=False),
        scratch_types=[
            pltpu.VMEM((GH, EPG), jnp.int32),
            pltpu.VMEM((GH, EPG), jnp.int32),
            pltpu.VMEM((4, EPG, 64), jnp.float32),
            pltpu.VMEM((ZR, 64), jnp.float32),
            pltpu.VMEM_SHARED((NP, 64), jnp.float32),
            pltpu.SemaphoreType.DMA,
            pltpu.SemaphoreType.DMA,
        ],
    )


_R = 1024
_RM = 512


def _l1_body(x_ref, w_ref, d0_ref, d1_ref, g_ref, dis_ref):
    deg = d0_ref[:, :1] + d1_ref[:, :1] + 1.0
    dis = lax.rsqrt(deg)
    dis_ref[...] = dis
    g_ref[...] = dis * jnp.dot(x_ref[...], w_ref[...],
                               preferred_element_type=jnp.float32)


def _mid_body(p_ref, g_ref, dis_ref, b_ref, w_ref, o_ref):
    dis = dis_ref[...]
    h = jnp.maximum(dis * (p_ref[...] + g_ref[...]) + b_ref[...], 0.0)
    o_ref[...] = dis * jnp.dot(h, w_ref[...],
                               preferred_element_type=jnp.float32)


def _midm_body(p_ref, g_ref, dis_ref, b_ref, w_ref, o_ref):
    dis = dis_ref[...]
    t = p_ref[0] + p_ref[1] + g_ref[...]
    h = jnp.maximum(dis * t + b_ref[...], 0.0)
    o_ref[...] = dis * jnp.dot(h, w_ref[...],
                               preferred_element_type=jnp.float32)


def _finm_body(p_ref, g_ref, dis_ref, b_ref, o_ref):
    t = p_ref[0] + p_ref[1] + g_ref[...]
    o_ref[...] = dis_ref[...] * t + b_ref[...]


def _row_spec(w):
    return pl.BlockSpec((_R, w), lambda i: (i, 0))


def _full_spec(a, b):
    return pl.BlockSpec((a, b), lambda i: (0, 0))


def _l1_call(x, w1, d0, d1):
    return pl.pallas_call(
        _l1_body,
        grid=(NP // _R,),
        in_specs=[_row_spec(128), _full_spec(128, 128),
                  _row_spec(LANES), _row_spec(LANES)],
        out_specs=[_row_spec(128), _row_spec(1)],
        out_shape=[jax.ShapeDtypeStruct((NP, 128), jnp.float32),
                   jax.ShapeDtypeStruct((NP, 1), jnp.float32)],
    )(x, w1, d0, d1)


def _mid_call(p, g, dis, b, w):
    wout = w.shape[1]
    return pl.pallas_call(
        _mid_body,
        grid=(NP // _R,),
        in_specs=[_row_spec(128), _row_spec(128), _row_spec(1),
                  _full_spec(1, 128), _full_spec(128, wout)],
        out_specs=_row_spec(wout),
        out_shape=jax.ShapeDtypeStruct((NP, wout), jnp.float32),
    )(p, g, dis, b, w)


def _midm_call(p, g, dism, bm, wbd):
    return pl.pallas_call(
        _midm_body,
        grid=(NM // _RM,),
        in_specs=[pl.BlockSpec((NC, _RM, 128), lambda i: (0, i, 0)),
                  pl.BlockSpec((_RM, 128), lambda i: (i, 0)),
                  pl.BlockSpec((_RM, 128), lambda i: (i, 0)),
                  _full_spec(1, 128), _full_spec(128, 128)],
        out_specs=pl.BlockSpec((_RM, 128), lambda i: (i, 0)),
        out_shape=jax.ShapeDtypeStruct((NM, 128), jnp.float32),
    )(p, g, dism, bm, wbd)


def _finm_call(p, g, dism, bm):
    return pl.pallas_call(
        _finm_body,
        grid=(NM // _RM,),
        in_specs=[pl.BlockSpec((NC, _RM, 128), lambda i: (0, i, 0)),
                  pl.BlockSpec((_RM, 128), lambda i: (i, 0)),
                  pl.BlockSpec((_RM, 128), lambda i: (i, 0)),
                  _full_spec(1, 128)],
        out_specs=pl.BlockSpec((_RM, 128), lambda i: (i, 0)),
        out_shape=jax.ShapeDtypeStruct((NM, 128), jnp.float32),
    )(p, g, dism, bm)


@jax.jit
def kernel(x, edge_index, W1, b1, W2, b2, W3, b3, W4, b4):
    src = edge_index[0]
    dst = edge_index[1]
    xp = jnp.concatenate([x, jnp.zeros((NP - N, x.shape[1]), x.dtype)])

    pad = NS * GS * EPG - E
    pad_src = (jnp.arange(pad, dtype=jnp.int32) * 37) % N
    pad_dst = N + jnp.arange(pad, dtype=jnp.int32) % (NP - N)
    src_s = jnp.concatenate([src, pad_src]).reshape(NS, GS, EPG)
    dst_s = jnp.concatenate([dst, pad_dst]).reshape(NS, GS, EPG)

    degp = _make_deg_kernel()(dst_s)
    edge128 = _make_edge128_kernel()
    edge64 = _make_edge64_kernel()

    g1, dis = _l1_call(xp, W1, degp[0], degp[1])
    dism = jnp.repeat(dis.reshape(NM, 2), 64, axis=1)
    z64 = jnp.zeros((64, 64), jnp.float32)
    w4bd = jnp.block([[W4, z64], [z64, W4]])
    b3m = jnp.concatenate([b3, b3]).reshape(1, 128)
    b4m = jnp.concatenate([b4, b4]).reshape(1, 128)

    p = edge128(g1.reshape(2 * NP, 64), src_s, dst_s)
    g2 = _mid_call(p, g1, dis, b1.reshape(1, -1), W2)
    p = edge128(g2.reshape(2 * NP, 64), src_s, dst_s)
    g3 = _mid_call(p, g2, dis, b2.reshape(1, -1), W3)
    g3m = g3.reshape(NM, 128)
    p = edge64(g3, src_s, dst_s)
    g4m = _midm_call(p.reshape(NC, NM, 128), g3m, dism, b3m, w4bd)
    p = edge64(g4m.reshape(NP, 64), src_s, dst_s)
    out_m = _finm_call(p.reshape(NC, NM, 128), g4m, dism, b4m)
    return out_m[:N // 2].reshape(N, 64)
